# Initial kernel scaffold; baseline (speedup 1.0000x reference)
import jax
import jax.numpy as jnp
from jax.experimental import pallas as pl


def kernel(x, dim, index, src):
    raise NotImplementedError("write your pallas kernel here")


# speedup vs baseline: 3.5471x; 3.5471x over previous
import functools

import jax
import jax.numpy as jnp
from jax import lax
from jax.experimental import pallas as pl
from jax.experimental.pallas import tpu as pltpu
from jax.experimental.pallas import tpu_sc as plsc
from jax._src.pallas import mpmd as _mpmd

NC = 2
NS = 16
L = 16

CH = 2048
SR = CH // 128


@functools.partial(jax.jit, static_argnums=(3, 4, 5))
def _sc_scatter(x_flat, idx_t, src_t, m, d, b):
  nw = NC * NS
  cols_per_w = d // nw
  nv = CH // L
  nchunk = b // CH
  init = jnp.int32(0x7FFFFFFF)

  mesh = plsc.VectorSubcoreMesh(
      core_axis_name="c", subcore_axis_name="s", num_cores=NC,
      num_subcores=NS)

  def body(x_ref, idx_ref, src_ref, out_ref, tag, srcbuf, ivbuf, flatbuf,
           valbuf, dsem):
    del x_ref
    w = lax.axis_index("s") * NC + lax.axis_index("c")

    def initb(i, _):
      tag[pl.ds(i * L, L)] = jnp.full((L,), init, jnp.int32)
      return 0
    lax.fori_loop(0, m // L, initb, 0)

    for lc in range(cols_per_w):
      col = w * cols_per_w + lc
      colbase = lc * b

      pltpu.sync_copy(src_ref.at[col], srcbuf)

      def chunk1(cidx, _):
        base = cidx * CH
        pltpu.sync_copy(idx_ref.at[col, pl.ds(base, CH)], ivbuf)

        def v1(k, _):
          iv = ivbuf[pl.ds(k * L, L)]
          gen = (colbase + base + k * L) + lax.iota(jnp.int32, L)
          plsc.store_scatter(tag, [iv], gen)
          t = plsc.load_gather(tag, [iv])

          def wcond(t_):
            return jnp.any(t_ < gen)

          def wbody(t_):
            plsc.store_scatter(tag, [iv], gen, mask=t_ < gen)
            return plsc.load_gather(tag, [iv])

          lax.while_loop(wcond, wbody, t)
          return 0
        lax.fori_loop(0, nv, v1, 0)
        return 0
      lax.fori_loop(0, nchunk, chunk1, 0)

      def chunk2(cidx, _):
        base = cidx * CH
        pltpu.sync_copy(idx_ref.at[col, pl.ds(base, CH)], ivbuf)

        def v2(k, _):
          iv = ivbuf[pl.ds(k * L, L)]
          t = plsc.load_gather(tag, [iv])
          vals = plsc.load_gather(srcbuf, [t - colbase])
          flat = iv * d + col
          r = k // 8
          o = (k % 8) * L
          flatbuf[r, pl.ds(o, L)] = flat
          valbuf[r, pl.ds(o, L)] = vals
          return 0
        lax.fori_loop(0, nv, v2, 0)

        descs = [
            pltpu.async_copy(valbuf.at[c], out_ref.at[flatbuf.at[c]], dsem)
            for c in range(SR)
        ]
        for dsc in descs:
          dsc.wait()
        return 0
      lax.fori_loop(0, nchunk, chunk2, 0)

  fn = _mpmd._mpmd_map(
      [(mesh, body)],
      jax.ShapeDtypeStruct((m * d,), jnp.float32),
      input_output_aliases={0: 0},
      compiler_params=pltpu.CompilerParams(needs_layout_passes=False),
      scratch_types=[
          pltpu.VMEM((m,), jnp.int32),
          pltpu.VMEM((b,), jnp.float32),
          pltpu.VMEM((CH,), jnp.int32),
          pltpu.VMEM((SR, 128), jnp.int32),
          pltpu.VMEM((SR, 128), jnp.float32),
          pltpu.SemaphoreType.DMA,
      ],
      name="scatter_overwrite_sc",
  )
  return fn(x_flat, idx_t, src_t)


def kernel(x, dim, index, src):
  m, d = x.shape
  b = src.shape[0]
  rows = (index + dim).astype(jnp.int32)
  idx_t = rows.T
  src_t = src.T
  out_flat = _sc_scatter(x.reshape(m * d), idx_t, src_t, m, d, b)
  return out_flat.reshape(m, d)
